# baseline (speedup 1.0000x reference)
import functools

import jax
import jax.numpy as jnp
from jax import lax
from jax.experimental import pallas as pl
from jax.experimental.pallas import tpu as pltpu
from jax.experimental.pallas import tpu_sc as plsc

_VB = 8192
_TAIL = 32768
_CHUNK = 64


def _gather_hid_body(nctx, s_ref, *refs):
    blk_refs = refs[:nctx]
    colx_ref, w1_ref, b1_ref, hid_ref = refs[nctx:]
    ext = jnp.concatenate([r[...] for r in blk_refs], axis=0)
    lanes = lax.broadcasted_iota(jnp.int32, ext.shape, 1)
    e_col = jnp.sum(jnp.where(lanes == colx_ref[...], ext, 0.0),
                    axis=1, keepdims=True)
    h_col = lax.dot_general(w1_ref[...], e_col, (((1,), (0,)), ((), ())),
                            preferred_element_type=jnp.float32)
    hid_ref[...] = jnp.maximum(h_col.T + b1_ref[...], 0.0)


def _tc_gather_hid(table_t, blkidx, colx, W1, b1):
    embd = table_t.shape[0]
    hid_dim = W1.shape[0]
    nctx = blkidx.shape[0]

    def make_map(j):
        return lambda i, s: (0, s[j])

    grid_spec = pltpu.PrefetchScalarGridSpec(
        num_scalar_prefetch=1,
        grid=(1,),
        in_specs=(
            [pl.BlockSpec((embd, 128), make_map(j)) for j in range(nctx)]
            + [
                pl.BlockSpec((nctx * embd, 1), lambda i, s: (0, 0)),
                pl.BlockSpec((hid_dim, nctx * embd), lambda i, s: (0, 0)),
                pl.BlockSpec((1, hid_dim), lambda i, s: (0, 0)),
            ]
        ),
        out_specs=pl.BlockSpec((1, hid_dim), lambda i, s: (0, 0)),
    )
    return pl.pallas_call(
        functools.partial(_gather_hid_body, nctx),
        grid_spec=grid_spec,
        out_shape=jax.ShapeDtypeStruct((1, hid_dim), jnp.float32),
    )(blkidx, *([table_t] * nctx), colx, W1, b1.reshape(1, hid_dim))


def _sc_tail_partials(W2, hid, start):
    hid_dim = W2.shape[1]
    ngrp = hid_dim // 16
    rows_per_w = _TAIL // 32
    nchunks = rows_per_w // _CHUNK
    mesh = plsc.VectorSubcoreMesh(core_axis_name="c", subcore_axis_name="s")

    @functools.partial(
        pl.kernel,
        out_type=jax.ShapeDtypeStruct((_TAIL * 16,), jnp.float32),
        mesh=mesh,
        scratch_types=[
            pltpu.VMEM((3, _CHUNK, hid_dim), jnp.float32),
            pltpu.VMEM((1, hid_dim), jnp.float32),
            pltpu.VMEM((rows_per_w * 16,), jnp.float32),
            pltpu.SemaphoreType.DMA((3,)),
            pltpu.SemaphoreType.DMA,
        ],
    )
    def tail_kernel(w2_hbm, hid_hbm, out_hbm, bufs, hid_v, pacc, sems, osem):
        w = lax.axis_index("c") * 16 + lax.axis_index("s")
        base = start + w * rows_per_w

        pltpu.sync_copy(hid_hbm, hid_v)
        hvecs = [hid_v[0, pl.ds(16 * g, 16)] for g in range(ngrp)]

        def start_dma(i, slot):
            pltpu.make_async_copy(
                w2_hbm.at[pl.ds(base + i * _CHUNK, _CHUNK), :],
                bufs.at[slot], sems.at[slot]).start()

        def wait_dma(slot):
            pltpu.make_async_copy(
                w2_hbm.at[pl.ds(base, _CHUNK), :],
                bufs.at[slot], sems.at[slot]).wait()

        start_dma(0, 0)
        start_dma(1, 1)

        def chunk_body(i, carry):
            slot = lax.rem(i, 3)
            wait_dma(slot)

            @pl.when(i + 2 < nchunks)
            def _():
                start_dma(i + 2, lax.rem(i + 2, 3))

            def row_body(r, carry2):
                acc = bufs[slot, r, pl.ds(0, 16)] * hvecs[0]
                for g in range(1, ngrp):
                    acc = acc + bufs[slot, r, pl.ds(16 * g, 16)] * hvecs[g]
                pacc[pl.ds((i * _CHUNK + r) * 16, 16)] = acc
                return carry2

            lax.fori_loop(0, _CHUNK, row_body, 0)
            return carry

        lax.fori_loop(0, nchunks, chunk_body, 0)
        c = pltpu.make_async_copy(
            pacc, out_hbm.at[pl.ds(w * rows_per_w * 16, rows_per_w * 16)],
            osem)
        c.start()
        c.wait()

    return tail_kernel(W2, hid)


def _head_body(nb, head, hid_ref, w2_ref, b2_ref, out_ref, m_ref, s_ref):
    i = pl.program_id(0)

    @pl.when(i == 0)
    def _():
        m_ref[...] = jnp.full((1, 1), -jnp.inf, jnp.float32)
        s_ref[...] = jnp.zeros((1, 1), jnp.float32)

    logits = lax.dot_general(hid_ref[...], w2_ref[...],
                             (((1,), (1,)), ((), ())),
                             preferred_element_type=jnp.float32)
    logits = logits + b2_ref[...]
    col = i * _VB + lax.broadcasted_iota(jnp.int32, (1, _VB), 1)
    logits = jnp.where(col < head, logits, -jnp.inf)
    out_ref[:, pl.ds(i * _VB, _VB)] = logits

    m_old = m_ref[...]
    m_new = jnp.maximum(m_old, jnp.max(logits, axis=(0, 1), keepdims=True))
    s_ref[...] = (s_ref[...] * jnp.exp(m_old - m_new)
                  + jnp.sum(jnp.exp(logits - m_new), axis=(0, 1), keepdims=True))
    m_ref[...] = m_new


def _tc_head(hid, W2, b2row, head):
    hid_dim = W2.shape[1]
    nb = -(-head // _VB)
    hpad = nb * _VB
    return pl.pallas_call(
        functools.partial(_head_body, nb, head),
        grid=(nb,),
        in_specs=[
            pl.BlockSpec((1, hid_dim), lambda i: (0, 0)),
            pl.BlockSpec((_VB, hid_dim), lambda i: (i, 0)),
            pl.BlockSpec((1, _VB), lambda i: (0, i)),
        ],
        out_specs=[
            pl.BlockSpec((1, hpad), lambda i: (0, 0)),
            pl.BlockSpec((1, 1), lambda i: (0, 0)),
            pl.BlockSpec((1, 1), lambda i: (0, 0)),
        ],
        out_shape=[
            jax.ShapeDtypeStruct((1, hpad), jnp.float32),
            jax.ShapeDtypeStruct((1, 1), jnp.float32),
            jax.ShapeDtypeStruct((1, 1), jnp.float32),
        ],
    )(hid, W2, b2row)


def _combine_body(head_ref, m1_ref, s1_ref, part_ref, b2t_ref, hout_ref,
                  tout_ref):
    lanec = lax.broadcasted_iota(jnp.int32, (128, 8), 0)
    grp = lax.broadcasted_iota(jnp.int32, (128, 8), 1)
    seg = jnp.where(lanec // 16 == grp, 1.0, 0.0)
    tail = lax.dot_general(part_ref[...], seg, (((1,), (0,)), ((), ())),
                           preferred_element_type=jnp.float32)
    tail = tail + b2t_ref[...]
    m2 = jnp.max(tail, axis=(0, 1), keepdims=True)
    s2 = jnp.sum(jnp.exp(tail - m2), axis=(0, 1), keepdims=True)
    m1 = m1_ref[...]
    s1 = s1_ref[...]
    m = jnp.maximum(m1, m2)
    s_all = s1 * jnp.exp(m1 - m) + s2 * jnp.exp(m2 - m)
    lse = (m + jnp.log(s_all))[0, 0]
    hout_ref[...] = head_ref[...] - lse
    tout_ref[...] = tail - lse


def _tc_combine(head_logits, m1, s1, partials, b2_tail):
    hpad = head_logits.shape[1]
    return pl.pallas_call(
        _combine_body,
        out_shape=[
            jax.ShapeDtypeStruct((1, hpad), jnp.float32),
            jax.ShapeDtypeStruct((_TAIL // 8, 8), jnp.float32),
        ],
    )(head_logits, m1, s1, partials, b2_tail)


def kernel(inputs, emb_table, W1, b1, W2, b2):
    vocab, embd = emb_table.shape
    hid_dim = W1.shape[0]
    nctx = inputs.shape[0]
    head = vocab - _TAIL

    idx = inputs.astype(jnp.int32)
    colx = jnp.repeat(idx % 128, embd).reshape(nctx * embd, 1)
    hid = _tc_gather_hid(emb_table.T, idx // 128, colx, W1, b1)

    partials = _sc_tail_partials(W2, hid, head)
    head_logits, m1, s1 = _tc_head(hid, W2, b2.reshape(1, vocab), head)
    b2_tail = b2[head:].reshape(_TAIL // 8, 8)
    hout, tout = _tc_combine(head_logits, m1, s1,
                             partials.reshape(_TAIL // 8, 128), b2_tail)

    return jnp.concatenate([hout[:, :head], tout.reshape(1, _TAIL)], axis=1)


# speedup vs baseline: 1.5088x; 1.5088x over previous
import functools

import jax
import jax.numpy as jnp
from jax import lax
from jax.experimental import pallas as pl
from jax.experimental.pallas import tpu as pltpu

_VB = 8192


def _fused_body(nctx, nb, vocab, s_ref, *refs):
    blk_refs = refs[:nctx]
    w1_ref, b1_ref, w2_ref, b2_ref, out_ref, hid_ref, m_ref, s_acc = refs[nctx:]
    i = pl.program_id(0)

    @pl.when(i == 0)
    def _():
        cols = []
        for j in range(nctx):
            blk = blk_refs[j][...]
            lanes = lax.broadcasted_iota(jnp.int32, blk.shape, 1)
            c = lax.rem(s_ref[j], 128)
            cols.append(jnp.sum(jnp.where(lanes == c, blk, 0.0),
                                axis=1, keepdims=True))
        e_col = jnp.concatenate(cols, axis=0)
        h_col = lax.dot_general(w1_ref[...], e_col, (((1,), (0,)), ((), ())),
                                preferred_element_type=jnp.float32)
        hid_ref[...] = jnp.maximum(h_col.T + b1_ref[...], 0.0)
        m_ref[...] = jnp.full((1, 1), -jnp.inf, jnp.float32)
        s_acc[...] = jnp.zeros((1, 1), jnp.float32)

    logits = lax.dot_general(hid_ref[...], w2_ref[...],
                             (((1,), (1,)), ((), ())),
                             preferred_element_type=jnp.float32)
    logits = logits + b2_ref[...]
    col = i * _VB + lax.broadcasted_iota(jnp.int32, (1, _VB), 1)
    logits = jnp.where(col < vocab, logits, -jnp.inf)
    out_ref[:, pl.ds(i * _VB, _VB)] = logits

    m_old = m_ref[...]
    m_new = jnp.maximum(m_old, jnp.max(logits, axis=(0, 1), keepdims=True))
    s_acc[...] = (s_acc[...] * jnp.exp(m_old - m_new)
                  + jnp.sum(jnp.exp(logits - m_new), axis=(0, 1), keepdims=True))
    m_ref[...] = m_new

    @pl.when(i == nb - 1)
    def _():
        lse = m_ref[...] + jnp.log(s_acc[...])
        out_ref[...] = out_ref[...] - lse[0, 0]


def kernel(inputs, emb_table, W1, b1, W2, b2):
    vocab, embd = emb_table.shape
    hid_dim = W1.shape[0]
    nctx = inputs.shape[0]
    nb = -(-vocab // _VB)
    vpad = nb * _VB

    def make_map(j):
        return lambda i, s: (0, s[j] // 128)

    grid_spec = pltpu.PrefetchScalarGridSpec(
        num_scalar_prefetch=1,
        grid=(nb,),
        in_specs=(
            [pl.BlockSpec((embd, 128), make_map(j)) for j in range(nctx)]
            + [
                pl.BlockSpec((hid_dim, nctx * embd), lambda i, s: (0, 0)),
                pl.BlockSpec((1, hid_dim), lambda i, s: (0, 0)),
                pl.BlockSpec((_VB, hid_dim), lambda i, s: (i, 0)),
                pl.BlockSpec((1, _VB), lambda i, s: (0, i)),
            ]
        ),
        out_specs=pl.BlockSpec((1, vpad), lambda i, s: (0, 0)),
        scratch_shapes=[
            pltpu.VMEM((1, hid_dim), jnp.float32),
            pltpu.VMEM((1, 1), jnp.float32),
            pltpu.VMEM((1, 1), jnp.float32),
        ],
    )
    out = pl.pallas_call(
        functools.partial(_fused_body, nctx, nb, vocab),
        grid_spec=grid_spec,
        out_shape=jax.ShapeDtypeStruct((1, vpad), jnp.float32),
    )(inputs.astype(jnp.int32), *([emb_table.T] * nctx), W1,
      b1.reshape(1, hid_dim), W2, b2.reshape(1, vocab))

    return out[:, :vocab]
